# baseline (speedup 1.0000x reference)
import functools
import numpy as np
import jax
import jax.numpy as jnp
from jax import lax
from jax.experimental import pallas as pl
from jax.experimental.pallas import tpu as pltpu
from jax.experimental.pallas import tpu_sc as plsc

ROWS = 16384
CLASSES = 1000
BLOCK_COLS = 2048
NBLK = ROWS // BLOCK_COLS
NUM_BINS = 15

_BOUNDS = np.linspace(0.0, 1.0, NUM_BINS + 1)

_SC_TILES = 16
_CHUNK = ROWS // _SC_TILES
_LANES = 16
_NVEC = _CHUNK // _LANES


def _rowstats_body(xt_ref, lab_ref, conf_hbm, acc_hbm, conf_v, acc_v,
                   sem_c, sem_a):
    i = pl.program_id(0)
    x = xt_ref[...]
    m = jnp.max(x, axis=0, keepdims=True)
    s = jnp.sum(jnp.exp(x - m), axis=0)
    idx = lax.broadcasted_iota(jnp.int32, x.shape, 0)
    pred = jnp.min(jnp.where(x == m, idx, CLASSES), axis=0)
    conf_v[...] = 1.0 / s
    acc_v[...] = (pred == lab_ref[0, 0, :]).astype(jnp.float32)
    cp_c = pltpu.make_async_copy(conf_v, conf_hbm.at[pl.ds(i * BLOCK_COLS, BLOCK_COLS)], sem_c)
    cp_a = pltpu.make_async_copy(acc_v, acc_hbm.at[pl.ds(i * BLOCK_COLS, BLOCK_COLS)], sem_a)
    cp_c.start()
    cp_a.start()
    cp_c.wait()
    cp_a.wait()


def _rowstats(xt, labs):
    return pl.pallas_call(
        _rowstats_body,
        grid=(NBLK,),
        in_specs=[
            pl.BlockSpec((CLASSES, BLOCK_COLS), lambda i: (0, i)),
            pl.BlockSpec((1, 1, BLOCK_COLS), lambda i: (i, 0, 0)),
        ],
        out_specs=[
            pl.BlockSpec(memory_space=pl.ANY),
            pl.BlockSpec(memory_space=pl.ANY),
        ],
        out_shape=[
            jax.ShapeDtypeStruct((ROWS,), jnp.float32),
            jax.ShapeDtypeStruct((ROWS,), jnp.float32),
        ],
        scratch_shapes=[
            pltpu.VMEM((BLOCK_COLS,), jnp.float32),
            pltpu.VMEM((BLOCK_COLS,), jnp.float32),
            pltpu.SemaphoreType.DMA,
            pltpu.SemaphoreType.DMA,
        ],
    )(xt, labs)


def _ece_sc_body(conf_hbm, acc_hbm, out_hbm, conf_v, acc_v, hist_v, part_v,
                 final_v, res_v, shared):
    sid = lax.axis_index("s")
    base = sid * _CHUNK
    pltpu.sync_copy(conf_hbm.at[pl.ds(base, _CHUNK)], conf_v)
    pltpu.sync_copy(acc_hbm.at[pl.ds(base, _CHUNK)], acc_v)

    zeros16 = jnp.zeros((_LANES,), jnp.float32)
    for i in range(3 * _LANES):
        hist_v[pl.ds(i * _LANES, _LANES)] = zeros16

    lane48 = lax.iota(jnp.int32, _LANES) * (3 * _LANES)
    ones16 = jnp.ones((_LANES,), jnp.float32)

    for j in range(_NVEC):
        c = conf_v[pl.ds(j * _LANES, _LANES)]
        a = acc_v[pl.ds(j * _LANES, _LANES)]
        b = jnp.minimum(
            (c * jnp.float32(NUM_BINS)).astype(jnp.int32), NUM_BINS - 1
        )
        plsc.addupdate_scatter(hist_v, [lane48 + b], ones16)
        plsc.addupdate_scatter(hist_v, [lane48 + (b + _LANES)], c)
        plsc.addupdate_scatter(hist_v, [lane48 + (b + 2 * _LANES)], a)

    for c in range(3):
        acc_vec = hist_v[pl.ds(c * _LANES, _LANES)]
        for l in range(1, _LANES):
            acc_vec = acc_vec + hist_v[pl.ds(l * (3 * _LANES) + c * _LANES, _LANES)]
        part_v[pl.ds(c * _LANES, _LANES)] = acc_vec

    pltpu.sync_copy(part_v, shared.at[pl.ds(sid * (3 * _LANES), 3 * _LANES)])
    plsc.subcore_barrier()

    @pl.when(sid == 0)
    def _():
        pltpu.sync_copy(shared, final_v)
        sums = []
        for c in range(3):
            acc_vec = final_v[pl.ds(c * _LANES, _LANES)]
            for l in range(1, _SC_TILES):
                acc_vec = acc_vec + final_v[pl.ds(l * (3 * _LANES) + c * _LANES, _LANES)]
            sums.append(acc_vec)
        counts, sum_conf, sum_acc = sums
        safe = jnp.maximum(counts, 1.0)
        contrib = jnp.where(
            counts > 0.0,
            jnp.abs(sum_conf / safe - sum_acc / safe) * (counts / float(ROWS)),
            0.0,
        )
        ece = jnp.sum(contrib)
        res_v[...] = jnp.full((_LANES,), ece, jnp.float32)
        pltpu.sync_copy(res_v, out_hbm)


_ece_sc = functools.partial(
    pl.kernel,
    out_type=jax.ShapeDtypeStruct((_LANES,), jnp.float32),
    mesh=plsc.VectorSubcoreMesh(
        core_axis_name="c", subcore_axis_name="s", num_cores=1
    ),
    compiler_params=pltpu.CompilerParams(needs_layout_passes=False),
    scratch_types=[
        pltpu.VMEM((_CHUNK,), jnp.float32),
        pltpu.VMEM((_CHUNK,), jnp.float32),
        pltpu.VMEM((_LANES * 3 * _LANES,), jnp.float32),
        pltpu.VMEM((3 * _LANES,), jnp.float32),
        pltpu.VMEM((_SC_TILES * 3 * _LANES,), jnp.float32),
        pltpu.VMEM((_LANES,), jnp.float32),
        pltpu.VMEM_SHARED((_SC_TILES * 3 * _LANES,), jnp.float32),
    ],
)(_ece_sc_body)


def kernel(inputs, labels):
    labs = labels.reshape(NBLK, 1, BLOCK_COLS)
    conf, acc = _rowstats(inputs.T, labs)
    ece16 = _ece_sc(conf, acc)
    return ece16[:1]


# speedup vs baseline: 1.0662x; 1.0662x over previous
import functools
import numpy as np
import jax
import jax.numpy as jnp
from jax import lax
from jax.experimental import pallas as pl
from jax.experimental.pallas import tpu as pltpu
from jax.experimental.pallas import tpu_sc as plsc

ROWS = 16384
CLASSES = 1000
BLOCK_COLS = 2048
NBLK = ROWS // BLOCK_COLS
NUM_BINS = 15

_BOUNDS = np.linspace(0.0, 1.0, NUM_BINS + 1)

_SC_TILES = 16
_CHUNK = ROWS // _SC_TILES
_LANES = 16
_NVEC = _CHUNK // _LANES


def _rowstats_body(xt_ref, lab_ref, conf_ref, acc_ref):
    x = xt_ref[...]
    m = jnp.max(x, axis=0, keepdims=True)
    s = jnp.sum(jnp.exp(x - m), axis=0)
    idx = lax.broadcasted_iota(jnp.int32, x.shape, 0)
    pred = jnp.min(jnp.where(x == m, idx, CLASSES), axis=0)
    conf_ref[0, 0, :] = 1.0 / s
    acc_ref[0, 0, :] = (pred == lab_ref[0, 0, :]).astype(jnp.float32)


def _rowstats(xt, labs):
    return pl.pallas_call(
        _rowstats_body,
        grid=(NBLK,),
        in_specs=[
            pl.BlockSpec((CLASSES, BLOCK_COLS), lambda i: (0, i)),
            pl.BlockSpec((1, 1, BLOCK_COLS), lambda i: (i, 0, 0)),
        ],
        out_specs=[
            pl.BlockSpec((1, 1, BLOCK_COLS), lambda i: (i, 0, 0)),
            pl.BlockSpec((1, 1, BLOCK_COLS), lambda i: (i, 0, 0)),
        ],
        out_shape=[
            jax.ShapeDtypeStruct((NBLK, 1, BLOCK_COLS), jnp.float32),
            jax.ShapeDtypeStruct((NBLK, 1, BLOCK_COLS), jnp.float32),
        ],
    )(xt, labs)


def _ece_sc_body(conf_hbm, acc_hbm, out_hbm, conf_v, acc_v, hist_v, part_v,
                 final_v, res_v, shared):
    sid = lax.axis_index("s")
    base = sid * _CHUNK
    pltpu.sync_copy(conf_hbm.at[pl.ds(base, _CHUNK)], conf_v)
    pltpu.sync_copy(acc_hbm.at[pl.ds(base, _CHUNK)], acc_v)

    zeros16 = jnp.zeros((_LANES,), jnp.float32)
    for i in range(3 * _LANES):
        hist_v[pl.ds(i * _LANES, _LANES)] = zeros16

    lane48 = lax.iota(jnp.int32, _LANES) * (3 * _LANES)
    ones16 = jnp.ones((_LANES,), jnp.float32)

    for j in range(_NVEC):
        c = conf_v[pl.ds(j * _LANES, _LANES)]
        a = acc_v[pl.ds(j * _LANES, _LANES)]
        b = jnp.minimum(
            (c * jnp.float32(NUM_BINS)).astype(jnp.int32), NUM_BINS - 1
        )
        plsc.addupdate_scatter(hist_v, [lane48 + b], ones16)
        plsc.addupdate_scatter(hist_v, [lane48 + (b + _LANES)], c)
        plsc.addupdate_scatter(hist_v, [lane48 + (b + 2 * _LANES)], a)

    for c in range(3):
        acc_vec = hist_v[pl.ds(c * _LANES, _LANES)]
        for l in range(1, _LANES):
            acc_vec = acc_vec + hist_v[pl.ds(l * (3 * _LANES) + c * _LANES, _LANES)]
        part_v[pl.ds(c * _LANES, _LANES)] = acc_vec

    pltpu.sync_copy(part_v, shared.at[pl.ds(sid * (3 * _LANES), 3 * _LANES)])
    plsc.subcore_barrier()

    @pl.when(sid == 0)
    def _():
        pltpu.sync_copy(shared, final_v)
        sums = []
        for c in range(3):
            acc_vec = final_v[pl.ds(c * _LANES, _LANES)]
            for l in range(1, _SC_TILES):
                acc_vec = acc_vec + final_v[pl.ds(l * (3 * _LANES) + c * _LANES, _LANES)]
            sums.append(acc_vec)
        counts, sum_conf, sum_acc = sums
        safe = jnp.maximum(counts, 1.0)
        contrib = jnp.where(
            counts > 0.0,
            jnp.abs(sum_conf / safe - sum_acc / safe) * (counts / float(ROWS)),
            0.0,
        )
        ece = jnp.sum(contrib)
        res_v[...] = jnp.full((_LANES,), ece, jnp.float32)
        pltpu.sync_copy(res_v, out_hbm)


_ece_sc = functools.partial(
    pl.kernel,
    out_type=jax.ShapeDtypeStruct((_LANES,), jnp.float32),
    mesh=plsc.VectorSubcoreMesh(
        core_axis_name="c", subcore_axis_name="s", num_cores=1
    ),
    compiler_params=pltpu.CompilerParams(needs_layout_passes=False),
    scratch_types=[
        pltpu.VMEM((_CHUNK,), jnp.float32),
        pltpu.VMEM((_CHUNK,), jnp.float32),
        pltpu.VMEM((_LANES * 3 * _LANES,), jnp.float32),
        pltpu.VMEM((3 * _LANES,), jnp.float32),
        pltpu.VMEM((_SC_TILES * 3 * _LANES,), jnp.float32),
        pltpu.VMEM((_LANES,), jnp.float32),
        pltpu.VMEM_SHARED((_SC_TILES * 3 * _LANES,), jnp.float32),
    ],
)(_ece_sc_body)


def kernel(inputs, labels):
    labs = labels.reshape(NBLK, 1, BLOCK_COLS)
    conf, acc = _rowstats(inputs.T, labs)
    ece16 = _ece_sc(conf.reshape(ROWS), acc.reshape(ROWS))
    return ece16[:1]
